# baseline (speedup 1.0000x reference)
import jax
import jax.numpy as jnp
from jax import lax
from jax.experimental import pallas as pl
from jax.experimental.pallas import tpu as pltpu, tpu_sc as plsc


def _sc_adj_body(ei_hbm, at_hbm, row_v, col_v, at_v):
    cid = lax.axis_index("c")
    sid = lax.axis_index("s")
    n = at_v.shape[0]
    e = ei_hbm.shape[1]

    @pl.when(jnp.logical_and(cid == 0, sid == 0))
    def _():
        pltpu.sync_copy(ei_hbm.at[0], row_v)
        pltpu.sync_copy(ei_hbm.at[1], col_v)
        zeros = jnp.zeros((16,), jnp.float32)
        for i in range(n):
            at_v[i, pl.ds(0, 16)] = zeros
            at_v[i, pl.ds(n - 16, 16)] = zeros
        ones = jnp.ones((16,), jnp.float32)
        lane = lax.iota(jnp.int32, 16)
        for i in range((e + 15) // 16):
            base = i * 16
            rem = e - base
            if rem >= 16:
                r = row_v[pl.ds(base, 16)]
                c = col_v[pl.ds(base, 16)]
                plsc.addupdate_scatter(at_v, [c, r], ones)
            else:
                r = row_v[pl.ds(e - 16, 16)]
                c = col_v[pl.ds(e - 16, 16)]
                mask = lane >= (16 - rem)
                plsc.addupdate_scatter(at_v, [c, r], ones, mask=mask)
        pltpu.sync_copy(at_v, at_hbm)


def _sc_adjacency(edge_index, n):
    e = edge_index.shape[1]
    mesh = plsc.VectorSubcoreMesh(core_axis_name="c", subcore_axis_name="s",
                                  num_cores=1)
    return pl.kernel(
        _sc_adj_body,
        out_type=jax.ShapeDtypeStruct((n, n), jnp.float32),
        mesh=mesh,
        compiler_params=pltpu.CompilerParams(needs_layout_passes=False),
        scratch_types=[
            pltpu.VMEM((e,), jnp.int32),
            pltpu.VMEM((e,), jnp.int32),
            pltpu.VMEM((n, n), jnp.float32),
        ],
    )(edge_index)


def _tc_dense_body(at_ref, x_ref, w1_ref, b1_ref, w2_ref, b2_ref, y_ref):
    w1 = w1_ref[...]
    norm1 = jnp.sqrt(jnp.sum(w1 * w1, axis=0, keepdims=True))
    scale1 = jnp.where(norm1 > 1.0, 1.0 / (norm1 + 1e-7), 1.0)
    h = lax.dot_general(
        x_ref[...] * scale1, w1, (((1,), (1,)), ((), ())),
        preferred_element_type=jnp.float32,
    )
    at = at_ref[...]
    deg = jnp.sum(at, axis=1, keepdims=True)
    dinv = jnp.where(deg > 0.0, lax.rsqrt(deg), 0.0)
    hs = h * dinv
    agg = lax.dot_general(
        at, hs, (((1,), (0,)), ((), ())),
        preferred_element_type=jnp.float32,
    ) * dinv
    a = agg + b1_ref[...]
    out = jnp.where(a > 0.0, a, jnp.exp(jnp.minimum(a, 0.0)) - 1.0)
    w2 = w2_ref[...]
    norm2 = jnp.sqrt(jnp.sum(w2 * w2, axis=1, keepdims=True))
    scale2 = jnp.where(norm2 > 0.5, 0.5 / (norm2 + 1e-7), 1.0)
    y_ref[...] = lax.dot_general(
        out, w2 * scale2, (((1,), (1,)), ((), ())),
        preferred_element_type=jnp.float32,
    ) + b2_ref[...]


def kernel(x, W1, b1, W2, b2, edge_index):
    n = x.shape[0]
    at = _sc_adjacency(edge_index.astype(jnp.int32), n)
    return pl.pallas_call(
        _tc_dense_body,
        out_shape=jax.ShapeDtypeStruct((n, W2.shape[0]), jnp.float32),
    )(at, x, W1, b1.reshape(1, -1), W2, b2.reshape(1, -1))


# speedup vs baseline: 1.0458x; 1.0458x over previous
import jax
import jax.numpy as jnp
from jax import lax
from jax.experimental import pallas as pl
from jax.experimental.pallas import tpu as pltpu, tpu_sc as plsc


def _sc_adj_body(ei_hbm, at_hbm, ei_s, at_s):
    cid = lax.axis_index("c")
    n = at_s.shape[0]
    e = ei_hbm.shape[1]

    @pl.when(cid == 0)
    def _():
        pltpu.sync_copy(ei_hbm, ei_s)

        def zero_row(i, _):
            def zero_col(j, _):
                at_s[i, j] = 0.0
                return ()
            return lax.fori_loop(0, n, zero_col, ())

        lax.fori_loop(0, n, zero_row, ())

        def scatter(k, _):
            r = ei_s[0, k]
            c = ei_s[1, k]
            at_s[c, r] = 1.0
            return ()

        lax.fori_loop(0, e, scatter, ())
        pltpu.sync_copy(at_s, at_hbm)


def _sc_adjacency(edge_index, n):
    e = edge_index.shape[1]
    mesh = plsc.ScalarSubcoreMesh(axis_name="c", num_cores=1)
    return pl.kernel(
        _sc_adj_body,
        out_type=jax.ShapeDtypeStruct((n, n), jnp.float32),
        mesh=mesh,
        compiler_params=pltpu.CompilerParams(needs_layout_passes=False),
        scratch_types=[
            pltpu.SMEM((2, e), jnp.int32),
            pltpu.SMEM((n, n), jnp.float32),
        ],
    )(edge_index)


def _tc_dense_body(at_ref, x_ref, w1_ref, b1_ref, w2_ref, b2_ref, y_ref):
    w1 = w1_ref[...]
    norm1 = jnp.sqrt(jnp.sum(w1 * w1, axis=0, keepdims=True))
    scale1 = jnp.where(norm1 > 1.0, 1.0 / (norm1 + 1e-7), 1.0)
    h = lax.dot_general(
        x_ref[...] * scale1, w1, (((1,), (1,)), ((), ())),
        preferred_element_type=jnp.float32,
    )
    at = at_ref[...]
    deg = jnp.sum(at, axis=1, keepdims=True)
    dinv = jnp.where(deg > 0.0, lax.rsqrt(deg), 0.0)
    hs = h * dinv
    agg = lax.dot_general(
        at, hs, (((1,), (0,)), ((), ())),
        preferred_element_type=jnp.float32,
    ) * dinv
    a = agg + b1_ref[...]
    out = jnp.where(a > 0.0, a, jnp.exp(jnp.minimum(a, 0.0)) - 1.0)
    w2 = w2_ref[...]
    norm2 = jnp.sqrt(jnp.sum(w2 * w2, axis=1, keepdims=True))
    scale2 = jnp.where(norm2 > 0.5, 0.5 / (norm2 + 1e-7), 1.0)
    y_ref[...] = lax.dot_general(
        out, w2 * scale2, (((1,), (1,)), ((), ())),
        preferred_element_type=jnp.float32,
    ) + b2_ref[...]


def kernel(x, W1, b1, W2, b2, edge_index):
    n = x.shape[0]
    at = _sc_adjacency(edge_index.astype(jnp.int32), n)
    return pl.pallas_call(
        _tc_dense_body,
        out_shape=jax.ShapeDtypeStruct((n, W2.shape[0]), jnp.float32),
    )(at, x, W1, b1.reshape(1, -1), W2, b2.reshape(1, -1))
